# baseline (speedup 1.0000x reference)
import functools

import jax
import jax.numpy as jnp
from jax.experimental import pallas as pl

MARGIN_ = 1.0
B_ = 4096
D_ = 16
CHUNK_ = 1024


def _triplet_body(x1_ref, x2_ref, x3_ref, out_ref):
    a = x1_ref[...]
    p = x2_ref[...]
    n = x3_ref[...]

    t = jnp.sum(p * (p - 2.0 * a), axis=1)

    ones = jnp.ones((B_, 1), jnp.float32)
    a_aug = jnp.concatenate([a, ones], axis=1)
    n2 = jnp.sum(n * n, axis=1, keepdims=True)
    n_aug = jnp.concatenate([-2.0 * n, n2], axis=1)

    acc = jnp.zeros((CHUNK_,), jnp.float32)
    for c in range(B_ // CHUNK_):
        rt = jax.lax.dot_general(
            n_aug, a_aug[c * CHUNK_:(c + 1) * CHUNK_],
            (((1,), (1,)), ((), ())),
            preferred_element_type=jnp.float32)
        m1 = jnp.min(rt.reshape(8, B_ // 8, CHUNK_), axis=0)
        m_c = jnp.min(m1, axis=0)
        t_c = t[c * CHUNK_:(c + 1) * CHUNK_]
        acc = acc + jnp.maximum(t_c - m_c + MARGIN_, 0.0)

    total = jnp.sum(acc)
    out_ref[...] = (total * jnp.float32(1.0 / B_)).reshape(1, 1)


@functools.partial(jax.jit, static_argnames=())
def kernel(x1, x2, x3):
    loss = pl.pallas_call(
        _triplet_body,
        out_shape=jax.ShapeDtypeStruct((1, 1), jnp.float32),
    )(x1, x2, x3)
    return (loss.reshape(()), jnp.asarray(B_, dtype=jnp.int32))


# speedup vs baseline: 1.0034x; 1.0034x over previous
import functools

import jax
import jax.numpy as jnp
from jax.experimental import pallas as pl

MARGIN_ = 1.0
B_ = 4096
D_ = 16
CHUNK_ = 256


def _triplet_body(x1_ref, x2_ref, x3_ref, out_ref):
    a = x1_ref[...]
    p = x2_ref[...]
    n = x3_ref[...]

    t = jnp.sum(p * (p - 2.0 * a), axis=1)

    ones = jnp.ones((B_, 1), jnp.float32)
    a_aug = jnp.concatenate([a, ones], axis=1)
    n2 = jnp.sum(n * n, axis=1, keepdims=True)
    n_aug = jnp.concatenate([-2.0 * n, n2], axis=1)

    acc = jnp.zeros((CHUNK_,), jnp.float32)
    for c in range(B_ // CHUNK_):
        rt = jax.lax.dot_general(
            n_aug, a_aug[c * CHUNK_:(c + 1) * CHUNK_],
            (((1,), (1,)), ((), ())),
            preferred_element_type=jnp.float32)
        m1 = jnp.min(rt.reshape(8, B_ // 8, CHUNK_), axis=0)
        m_c = jnp.min(m1, axis=0)
        t_c = t[c * CHUNK_:(c + 1) * CHUNK_]
        acc = acc + jnp.maximum(t_c - m_c + MARGIN_, 0.0)

    total = jnp.sum(acc)
    out_ref[...] = (total * jnp.float32(1.0 / B_)).reshape(1, 1)


@functools.partial(jax.jit, static_argnames=())
def kernel(x1, x2, x3):
    loss = pl.pallas_call(
        _triplet_body,
        out_shape=jax.ShapeDtypeStruct((1, 1), jnp.float32),
    )(x1, x2, x3)
    return (loss.reshape(()), jnp.asarray(B_, dtype=jnp.int32))
